# baseline (speedup 1.0000x reference)
import jax
import jax.numpy as jnp
from jax.experimental import pallas as pl
from jax.experimental.pallas import tpu as pltpu

ROWS = 128
COLS = 32768
RB = 8
NCHUNK = ROWS // RB
NBUF = 6
PRIME = 5


def _body(in_ref, out_ref, *scratch):
    bufs = list(scratch[:NBUF])
    sems = scratch[NBUF]

    def copy(k):
        return pltpu.make_async_copy(
            in_ref.at[pl.ds(k * RB, RB)], bufs[k % NBUF], sems.at[k % NBUF]
        )

    for k in range(PRIME):
        copy(k).start()

    idxs = []
    for k in range(NCHUNK):
        if k + PRIME < NCHUNK:
            copy(k + PRIME).start()
        copy(k).wait()
        a = jnp.argmax(bufs[k % NBUF][...], axis=1)
        idxs.append(a.reshape(RB, 1).astype(jnp.float32))

    idx_f = jnp.concatenate(idxs, axis=0)
    out_ref[...] = jnp.transpose(idx_f).astype(jnp.int32)


def kernel(inputs):
    out = pl.pallas_call(
        _body,
        in_specs=[pl.BlockSpec(memory_space=pl.ANY)],
        out_specs=pl.BlockSpec(memory_space=pltpu.VMEM),
        out_shape=jax.ShapeDtypeStruct((1, ROWS), jnp.int32),
        scratch_shapes=[pltpu.VMEM((RB, COLS), jnp.float32)] * NBUF
        + [pltpu.SemaphoreType.DMA((NBUF,))],
    )(inputs)
    return out.reshape(ROWS)


# speedup vs baseline: 1.1642x; 1.1642x over previous
import jax
import jax.numpy as jnp
from jax.experimental import pallas as pl
from jax.experimental.pallas import tpu as pltpu

ROWS = 128
COLS = 32768
RB = 16
NCHUNK = ROWS // RB
NBUF = 6
PRIME = 5


def _body(in_ref, out_ref, *scratch):
    bufs = list(scratch[:NBUF])
    sems = scratch[NBUF]

    def copy(k):
        return pltpu.make_async_copy(
            in_ref.at[pl.ds(k * RB, RB)], bufs[k % NBUF], sems.at[k % NBUF]
        )

    for k in range(PRIME):
        copy(k).start()

    idxs = []
    for k in range(NCHUNK):
        if k + PRIME < NCHUNK:
            copy(k + PRIME).start()
        copy(k).wait()
        a = jnp.argmax(bufs[k % NBUF][...], axis=1)
        idxs.append(a.reshape(RB, 1).astype(jnp.float32))

    idx_f = jnp.concatenate(idxs, axis=0)
    out_ref[...] = jnp.transpose(idx_f).astype(jnp.int32)


def kernel(inputs):
    out = pl.pallas_call(
        _body,
        in_specs=[pl.BlockSpec(memory_space=pl.ANY)],
        out_specs=pl.BlockSpec(memory_space=pltpu.VMEM),
        out_shape=jax.ShapeDtypeStruct((1, ROWS), jnp.int32),
        scratch_shapes=[pltpu.VMEM((RB, COLS), jnp.float32)] * NBUF
        + [pltpu.SemaphoreType.DMA((NBUF,))],
    )(inputs)
    return out.reshape(ROWS)


# speedup vs baseline: 1.1698x; 1.0048x over previous
import jax
import jax.numpy as jnp
from jax.experimental import pallas as pl
from jax.experimental.pallas import tpu as pltpu

ROWS = 128
COLS = 32768
RB = 32
NCHUNK = ROWS // RB
NBUF = 3
PRIME = 2


def _body(in_ref, out_ref, *scratch):
    bufs = list(scratch[:NBUF])
    sems = scratch[NBUF]

    def copy(k):
        return pltpu.make_async_copy(
            in_ref.at[pl.ds(k * RB, RB)], bufs[k % NBUF], sems.at[k % NBUF]
        )

    for k in range(PRIME):
        copy(k).start()

    idxs = []
    for k in range(NCHUNK):
        if k + PRIME < NCHUNK:
            copy(k + PRIME).start()
        copy(k).wait()
        a = jnp.argmax(bufs[k % NBUF][...], axis=1)
        idxs.append(a.reshape(RB, 1).astype(jnp.float32))

    idx_f = jnp.concatenate(idxs, axis=0)
    out_ref[...] = jnp.transpose(idx_f).astype(jnp.int32)


def kernel(inputs):
    out = pl.pallas_call(
        _body,
        in_specs=[pl.BlockSpec(memory_space=pl.ANY)],
        out_specs=pl.BlockSpec(memory_space=pltpu.VMEM),
        out_shape=jax.ShapeDtypeStruct((1, ROWS), jnp.int32),
        scratch_shapes=[pltpu.VMEM((RB, COLS), jnp.float32)] * NBUF
        + [pltpu.SemaphoreType.DMA((NBUF,))],
    )(inputs)
    return out.reshape(ROWS)
